# baseline (speedup 1.0000x reference)
import functools
import math

import jax
import jax.numpy as jnp
from jax import lax
from jax.experimental import pallas as pl
from jax.experimental.pallas import tpu as pltpu
from jax.experimental.pallas import tpu_sc as plsc

N = 10000
E = 160000
C = 256
NUM_RBF = 16
CUTOFF = 10.0

ALPHA1 = 1.0 / math.sqrt(3.0 * C)
SH0C = 1.0 / (2.0 * math.sqrt(math.pi))
SH1C = math.sqrt(3.0 / (4.0 * math.pi))

TN = 400
TE = 800

SRC_W = 2048
DST_W = 272
POS_W = 16
MSG_W = 1024

NC = 2
NS = 16
NW = NC * NS
EPW = E // NW
GW = 40
NWIN = EPW // GW
NPAD = 10240
NCHUNK = NPAD // NS
NGRP = MSG_W // 128


def _k1_body(xs_ref, xvT_ref, pos_ref, wsc_ref, wv_ref, src_ref, dst_ref, posp_ref):
    xs = xs_ref[...]
    sc = jnp.dot(xs, wsc_ref[...], preferred_element_type=jnp.float32)
    vparts = []
    for m in range(3):
        vm = jnp.dot(xvT_ref[m], wv_ref[...], preferred_element_type=jnp.float32)
        vparts.append(vm)
    src_ref[...] = jnp.concatenate(
        [sc[:, :512], vparts[0][:, :256], vparts[1][:, :256], vparts[2][:, :256],
         vparts[0][:, 256:], vparts[1][:, 256:], vparts[2][:, 256:]], axis=1)
    pos = pos_ref[...]
    dst_ref[...] = jnp.concatenate([sc[:, 512:768], pos], axis=1)
    posp_ref[...] = pos


def _node_tables(xs, xvT, pos16, Wsc, Wv):
    grid = (N // TN,)
    return pl.pallas_call(
        _k1_body,
        grid=grid,
        in_specs=[
            pl.BlockSpec((TN, C), lambda i: (i, 0)),
            pl.BlockSpec((3, TN, C), lambda i: (0, i, 0)),
            pl.BlockSpec((TN, POS_W), lambda i: (i, 0)),
            pl.BlockSpec((C, 768), lambda i: (0, 0)),
            pl.BlockSpec((C, 512), lambda i: (0, 0)),
        ],
        out_specs=[
            pl.BlockSpec((TN, SRC_W), lambda i: (i, 0)),
            pl.BlockSpec((TN, DST_W), lambda i: (i, 0)),
            pl.BlockSpec((TN, POS_W), lambda i: (i, 0)),
        ],
        out_shape=[
            jax.ShapeDtypeStruct((N, SRC_W), jnp.float32),
            jax.ShapeDtypeStruct((N, DST_W), jnp.float32),
            jax.ShapeDtypeStruct((N, POS_W), jnp.float32),
        ],
    )(xs, xvT, pos16, Wsc, Wv)


def _k4_body(srcg_ref, dpg_ref, prg_ref, w1r_ref, b1_ref, cen_ref, wid_ref, msg_ref):
    posr = prg_ref[:, :3]
    posc = dpg_ref[:, C:C + 3]
    diff = posr - posc
    dd = jnp.sum(diff * diff, axis=1, keepdims=True)
    dist = jnp.sqrt(dd)
    d = jnp.minimum(dist, CUTOFF)
    z = (d - cen_ref[...]) / wid_ref[...]
    rbf = jnp.exp(-(z * z)) * (1.0 - (d / CUTOFF) ** 2)
    rbfw = jnp.dot(rbf, w1r_ref[...], preferred_element_type=jnp.float32)
    pre = srcg_ref[:, :C] + dpg_ref[:, :C] + rbfw + b1_ref[...]
    h = pre * (1.0 / (1.0 + jnp.exp(-pre)))
    sh1 = SH1C * diff / (dist + 1e-8)
    sP = srcg_ref[:, C:2 * C]
    outs = [h]
    for m in range(3):
        m1, m2 = (m + 1) % 3, (m + 2) % 3
        vPm = srcg_ref[:, 512 + C * m:512 + C * (m + 1)]
        vQ1 = srcg_ref[:, 1280 + C * m1:1280 + C * (m1 + 1)]
        vQ2 = srcg_ref[:, 1280 + C * m2:1280 + C * (m2 + 1)]
        om = (sP * sh1[:, m:m + 1] + vPm
              + vQ1 * sh1[:, m2:m2 + 1] - vQ2 * sh1[:, m1:m1 + 1])
        outs.append(om)
    msg_ref[...] = jnp.concatenate(outs, axis=1)


def _edge_stage(srcg, dpg, prg, W1r, b1r, cen, wid):
    grid = (E // TE,)
    return pl.pallas_call(
        _k4_body,
        grid=grid,
        in_specs=[
            pl.BlockSpec((TE, SRC_W), lambda i: (i, 0)),
            pl.BlockSpec((TE, DST_W), lambda i: (i, 0)),
            pl.BlockSpec((TE, POS_W), lambda i: (i, 0)),
            pl.BlockSpec((NUM_RBF, C), lambda i: (0, 0)),
            pl.BlockSpec((1, C), lambda i: (0, 0)),
            pl.BlockSpec((1, NUM_RBF), lambda i: (0, 0)),
            pl.BlockSpec((1, NUM_RBF), lambda i: (0, 0)),
        ],
        out_specs=pl.BlockSpec((TE, MSG_W), lambda i: (i, 0)),
        out_shape=jax.ShapeDtypeStruct((E, MSG_W), jnp.float32),
    )(srcg, dpg, prg, W1r, b1r, cen, wid)


_SC_MESH = plsc.VectorSubcoreMesh(core_axis_name="c", subcore_axis_name="s")


def _k3_body(src_hbm, dst_hbm, pos_hbm, row_hbm, col_hbm,
             srcg_hbm, dpg_hbm, prg_hbm,
             idxr_v, idxc_v, buf_s, buf_d, buf_p, sem1, sem2, sem3):
    c = lax.axis_index("c")
    s = lax.axis_index("s")
    wid = s * NC + c
    base0 = wid * EPW

    def win(w, carry):
        base = base0 + w * GW
        pltpu.sync_copy(row_hbm.at[pl.ds(base, GW)], idxr_v)
        pltpu.sync_copy(col_hbm.at[pl.ds(base, GW)], idxc_v)
        ca = pltpu.async_copy(src_hbm.at[idxr_v], buf_s, sem1)
        cb = pltpu.async_copy(dst_hbm.at[idxc_v], buf_d, sem2)
        cc = pltpu.async_copy(pos_hbm.at[idxr_v], buf_p, sem3)
        ca.wait()
        cb.wait()
        cc.wait()
        pltpu.sync_copy(buf_s, srcg_hbm.at[pl.ds(base, GW)])
        pltpu.sync_copy(buf_d, dpg_hbm.at[pl.ds(base, GW)])
        pltpu.sync_copy(buf_p, prg_hbm.at[pl.ds(base, GW)])
        return carry

    lax.fori_loop(0, NWIN, win, 0)


_k3_call = functools.partial(
    pl.kernel,
    out_type=[
        jax.ShapeDtypeStruct((E, SRC_W), jnp.float32),
        jax.ShapeDtypeStruct((E, DST_W), jnp.float32),
        jax.ShapeDtypeStruct((E, POS_W), jnp.float32),
    ],
    mesh=_SC_MESH,
    scratch_types=[
        pltpu.VMEM((GW,), jnp.int32),
        pltpu.VMEM((GW,), jnp.int32),
        pltpu.VMEM((GW, SRC_W), jnp.float32),
        pltpu.VMEM((GW, DST_W), jnp.float32),
        pltpu.VMEM((GW, POS_W), jnp.float32),
        pltpu.SemaphoreType.DMA,
        pltpu.SemaphoreType.DMA,
        pltpu.SemaphoreType.DMA,
    ],
    compiler_params=pltpu.CompilerParams(use_tc_tiling_on_sc=False),
)(_k3_body)


def _k5_body(msgs_hbm, col_hbm, zeros_hbm, part_hbm, idx_v, vals_v, acc):
    c = lax.axis_index("c")
    s = lax.axis_index("s")
    wid = s * NC + c
    base0 = wid * EPW

    for g in range(NGRP):
        pltpu.sync_copy(zeros_hbm, acc.at[pl.ds(s * NCHUNK, NCHUNK)])
        plsc.subcore_barrier()

        def win(w, carry):
            base = base0 + w * GW
            pltpu.sync_copy(col_hbm.at[pl.ds(base, GW)], idx_v)
            pltpu.sync_copy(
                msgs_hbm.at[pl.ds(base, GW), pl.ds(g * 128, 128)], vals_v)
            pltpu.sync_copy(vals_v, acc.at[idx_v], add=True)
            return carry

        lax.fori_loop(0, NWIN, win, 0)
        plsc.subcore_barrier()
        pltpu.sync_copy(
            acc.at[pl.ds(s * NCHUNK, NCHUNK)],
            part_hbm.at[c, pl.ds(s * NCHUNK, NCHUNK), pl.ds(g * 128, 128)])
        plsc.subcore_barrier()


_k5_call = functools.partial(
    pl.kernel,
    out_type=jax.ShapeDtypeStruct((NC, NPAD, MSG_W), jnp.float32),
    mesh=_SC_MESH,
    scratch_types=[
        pltpu.VMEM((GW,), jnp.int32),
        pltpu.VMEM((GW, 128), jnp.float32),
        pltpu.VMEM_SHARED((NPAD, 128), jnp.float32),
    ],
)(_k5_body)


def _k6_body(m_ref, xs_ref, xvT_ref, w2_ref, b2_ref, outs_ref, outv_ref):
    m_ref = m_ref[0] + m_ref[1]
    H = m_ref[:, :C]
    so = jnp.dot(H, w2_ref[...], preferred_element_type=jnp.float32) + b2_ref[...]
    so = so * (1.0 / (1.0 + jnp.exp(-so)))
    gates = 1.0 / (1.0 + jnp.exp(-so))
    outs_ref[...] = xs_ref[...] + so
    for m in range(3):
        outv_ref[m] = xvT_ref[m] + m_ref[:, C * (m + 1):C * (m + 2)] * gates


def _finish(parts, xs, xvT, W2, b2r):
    grid = (N // TN,)
    return pl.pallas_call(
        _k6_body,
        grid=grid,
        in_specs=[
            pl.BlockSpec((NC, TN, MSG_W), lambda i: (0, i, 0)),
            pl.BlockSpec((TN, C), lambda i: (i, 0)),
            pl.BlockSpec((3, TN, C), lambda i: (0, i, 0)),
            pl.BlockSpec((C, C), lambda i: (0, 0)),
            pl.BlockSpec((1, C), lambda i: (0, 0)),
        ],
        out_specs=[
            pl.BlockSpec((TN, C), lambda i: (i, 0)),
            pl.BlockSpec((3, TN, C), lambda i: (0, i, 0)),
        ],
        out_shape=[
            jax.ShapeDtypeStruct((N, C), jnp.float32),
            jax.ShapeDtypeStruct((3, N, C), jnp.float32),
        ],
    )(parts, xs, xvT, W2, b2r)


@jax.jit
def kernel(x_scalar, x_vector, edge_index, edge_attr, pos, W1, b1, W2, b2,
           w00_0, w11_0, w01_1, w10_1, w11_1, centers, widths):
    del edge_attr, w00_0, w11_0
    xs = x_scalar
    xvT = jnp.transpose(x_vector, (2, 0, 1))
    pos16 = jnp.pad(pos, ((0, 0), (0, POS_W - 3)))
    row = edge_index[0]
    col = edge_index[1]

    Wsc = jnp.concatenate([W1[:C], ALPHA1 * w01_1, W1[C:2 * C]], axis=1)
    Wv = jnp.concatenate([(ALPHA1 * SH0C) * w10_1,
                          (ALPHA1 / math.sqrt(2.0)) * w11_1], axis=1)
    W1r = W1[2 * C:]
    b1r = b1.reshape(1, C)
    b2r = b2.reshape(1, C)
    cen = centers.reshape(1, NUM_RBF)
    wid = widths.reshape(1, NUM_RBF)

    src_tab, dst_tab, pos_tab = _node_tables(xs, xvT, pos16, Wsc, Wv)

    srcg, dpg, prg = _k3_call(src_tab, dst_tab, pos_tab, row, col)

    msgs = _edge_stage(srcg, dpg, prg, W1r, b1r, cen, wid)

    zeros = jnp.zeros((NCHUNK, 128), jnp.float32)
    parts = _k5_call(msgs, col, zeros)

    out_s, outvT = _finish(parts, xs, xvT, W2, b2r)
    out_v = jnp.transpose(outvT, (1, 2, 0))
    return (out_s, out_v)


# speedup vs baseline: 1.0439x; 1.0439x over previous
import functools
import math

import jax
import jax.numpy as jnp
from jax import lax
from jax.experimental import pallas as pl
from jax.experimental.pallas import tpu as pltpu
from jax.experimental.pallas import tpu_sc as plsc

N = 10000
E = 160000
C = 256
NUM_RBF = 16
CUTOFF = 10.0

ALPHA1 = 1.0 / math.sqrt(3.0 * C)
SH0C = 1.0 / (2.0 * math.sqrt(math.pi))
SH1C = math.sqrt(3.0 / (4.0 * math.pi))

TN = 400
TE = 800

SRC_W = 2048
DST_W = 272
POS_W = 16
MSG_W = 1024

NC = 2
NS = 16
NW = NC * NS
EPW = E // NW
GW = 40
NWIN = EPW // GW
NPAIR = (NWIN - 1) // 2
NPAD = 10240
NCHUNK = NPAD // NS
NGRP = MSG_W // 128


def _k1_body(xs_ref, xvT_ref, pos_ref, wsc_ref, wv_ref, src_ref, dst_ref, posp_ref):
    xs = xs_ref[...]
    sc = jnp.dot(xs, wsc_ref[...], preferred_element_type=jnp.float32)
    vparts = []
    for m in range(3):
        vm = jnp.dot(xvT_ref[m], wv_ref[...], preferred_element_type=jnp.float32)
        vparts.append(vm)
    src_ref[...] = jnp.concatenate(
        [sc[:, :512], vparts[0][:, :256], vparts[1][:, :256], vparts[2][:, :256],
         vparts[0][:, 256:], vparts[1][:, 256:], vparts[2][:, 256:]],
        axis=1).astype(jnp.bfloat16)
    pos = pos_ref[...]
    dst_ref[...] = jnp.concatenate([sc[:, 512:768], pos], axis=1)
    posp_ref[...] = pos


def _node_tables(xs, xvT, pos16, Wsc, Wv):
    grid = (N // TN,)
    return pl.pallas_call(
        _k1_body,
        grid=grid,
        in_specs=[
            pl.BlockSpec((TN, C), lambda i: (i, 0)),
            pl.BlockSpec((3, TN, C), lambda i: (0, i, 0)),
            pl.BlockSpec((TN, POS_W), lambda i: (i, 0)),
            pl.BlockSpec((C, 768), lambda i: (0, 0)),
            pl.BlockSpec((C, 512), lambda i: (0, 0)),
        ],
        out_specs=[
            pl.BlockSpec((TN, SRC_W), lambda i: (i, 0)),
            pl.BlockSpec((TN, DST_W), lambda i: (i, 0)),
            pl.BlockSpec((TN, POS_W), lambda i: (i, 0)),
        ],
        out_shape=[
            jax.ShapeDtypeStruct((N, SRC_W), jnp.bfloat16),
            jax.ShapeDtypeStruct((N, DST_W), jnp.float32),
            jax.ShapeDtypeStruct((N, POS_W), jnp.float32),
        ],
    )(xs, xvT, pos16, Wsc, Wv)


_SC_MESH = plsc.VectorSubcoreMesh(core_axis_name="c", subcore_axis_name="s")


def _k3_body(src_hbm, dst_hbm, pos_hbm, row2_hbm, col2_hbm,
             srcg_hbm, dpg_hbm, prg_hbm,
             idxr_v, idxc_v, buf_s, buf_d, buf_p, sem_s, sem_d, sem_p):
    c = lax.axis_index("c")
    s = lax.axis_index("s")
    wid = s * NC + c
    base0 = wid * EPW
    pltpu.sync_copy(row2_hbm.at[wid], idxr_v)
    pltpu.sync_copy(col2_hbm.at[wid], idxc_v)

    def _issue(slot, w):
        lb = w * GW
        pltpu.async_copy(src_hbm.at[idxr_v.at[pl.ds(lb, GW)]], buf_s.at[slot],
                         sem_s.at[slot])
        pltpu.async_copy(dst_hbm.at[idxc_v.at[pl.ds(lb, GW)]], buf_d.at[slot],
                         sem_d.at[slot])
        pltpu.async_copy(pos_hbm.at[idxr_v.at[pl.ds(lb, GW)]], buf_p.at[slot],
                         sem_p.at[slot])

    def _drain(slot):
        pltpu.make_async_copy(src_hbm.at[pl.ds(0, GW)], buf_s.at[slot],
                              sem_s.at[slot]).wait()
        pltpu.make_async_copy(dst_hbm.at[pl.ds(0, GW)], buf_d.at[slot],
                              sem_d.at[slot]).wait()
        pltpu.make_async_copy(pos_hbm.at[pl.ds(0, GW)], buf_p.at[slot],
                              sem_p.at[slot]).wait()

    def _write(slot, w):
        base = base0 + w * GW
        pltpu.sync_copy(buf_s.at[slot], srcg_hbm.at[pl.ds(base, GW)])
        pltpu.sync_copy(buf_d.at[slot], dpg_hbm.at[pl.ds(base, GW)])
        pltpu.sync_copy(buf_p.at[slot], prg_hbm.at[pl.ds(base, GW)])

    _issue(0, 0)

    def pair(p, carry):
        for b in range(2):
            w = p * 2 + b
            _issue(1 - b, w + 1)
            _drain(b)
            _write(b, w)
        return carry

    lax.fori_loop(0, NPAIR, pair, 0)
    _drain(0)
    _write(0, NWIN - 1)


_k3_call = functools.partial(
    pl.kernel,
    out_type=[
        jax.ShapeDtypeStruct((E, SRC_W), jnp.bfloat16),
        jax.ShapeDtypeStruct((E, DST_W), jnp.float32),
        jax.ShapeDtypeStruct((E, POS_W), jnp.float32),
    ],
    mesh=_SC_MESH,
    scratch_types=[
        pltpu.VMEM((EPW,), jnp.int32),
        pltpu.VMEM((EPW,), jnp.int32),
        pltpu.VMEM((2, GW, SRC_W), jnp.bfloat16),
        pltpu.VMEM((2, GW, DST_W), jnp.float32),
        pltpu.VMEM((2, GW, POS_W), jnp.float32),
        pltpu.SemaphoreType.DMA((2,)),
        pltpu.SemaphoreType.DMA((2,)),
        pltpu.SemaphoreType.DMA((2,)),
    ],
    compiler_params=pltpu.CompilerParams(use_tc_tiling_on_sc=False),
)(_k3_body)


def _k5_body(msgs_hbm, col3_hbm, zeros_hbm, part_hbm, colw_v, vals_v, acc, sem_v):
    c = lax.axis_index("c")
    s = lax.axis_index("s")
    wid = s * NC + c
    base0 = wid * EPW
    pltpu.sync_copy(col3_hbm.at[wid], colw_v)

    for g in range(NGRP):
        pltpu.sync_copy(zeros_hbm, acc.at[pl.ds(s * NCHUNK, NCHUNK)])
        plsc.subcore_barrier()

        def _issue(slot, w):
            pltpu.async_copy(msgs_hbm.at[g, pl.ds(base0 + w * GW, GW)],
                             vals_v.at[slot], sem_v.at[slot])

        def _drain(slot):
            pltpu.make_async_copy(msgs_hbm.at[g, pl.ds(0, GW)],
                                  vals_v.at[slot], sem_v.at[slot]).wait()

        def _scat(slot, w):
            pltpu.sync_copy(vals_v.at[slot], acc.at[colw_v.at[w]], add=True)

        _issue(0, 0)

        def pair(p, carry):
            for b in range(2):
                w = p * 2 + b
                _issue(1 - b, w + 1)
                _drain(b)
                _scat(b, w)
            return carry

        lax.fori_loop(0, NPAIR, pair, 0)
        _drain(0)
        _scat(0, NWIN - 1)
        plsc.subcore_barrier()
        pltpu.sync_copy(acc.at[pl.ds(s * NCHUNK, NCHUNK)],
                        part_hbm.at[c, g, pl.ds(s * NCHUNK, NCHUNK)])
        plsc.subcore_barrier()


_k5_call = functools.partial(
    pl.kernel,
    out_type=jax.ShapeDtypeStruct((NC, NGRP, NPAD, 128), jnp.float32),
    mesh=_SC_MESH,
    scratch_types=[
        pltpu.VMEM((NWIN, GW), jnp.int32),
        pltpu.VMEM((2, GW, 128), jnp.float32),
        pltpu.VMEM_SHARED((NPAD, 128), jnp.float32),
        pltpu.SemaphoreType.DMA((2,)),
    ],
    compiler_params=pltpu.CompilerParams(use_tc_tiling_on_sc=False),
)(_k5_body)


def _k4_body(srcg_ref, dpg_ref, prg_ref, w1r_ref, b1_ref, cen_ref, wid_ref, msg_ref):
    posr = prg_ref[:, :3]
    posc = dpg_ref[:, C:C + 3]
    diff = posr - posc
    dd = jnp.sum(diff * diff, axis=1, keepdims=True)
    dist = jnp.sqrt(dd)
    d = jnp.minimum(dist, CUTOFF)
    z = (d - cen_ref[...]) / wid_ref[...]
    rbf = jnp.exp(-(z * z)) * (1.0 - (d / CUTOFF) ** 2)
    rbfw = jnp.dot(rbf, w1r_ref[...], preferred_element_type=jnp.float32)
    A = srcg_ref[:, :C].astype(jnp.float32)
    pre = A + dpg_ref[:, :C] + rbfw + b1_ref[...]
    h = pre * (1.0 / (1.0 + jnp.exp(-pre)))
    sh1 = SH1C * diff / (dist + 1e-8)
    sP = srcg_ref[:, C:2 * C].astype(jnp.float32)
    outs = [h]
    for m in range(3):
        m1, m2 = (m + 1) % 3, (m + 2) % 3
        vPm = srcg_ref[:, 512 + C * m:512 + C * (m + 1)].astype(jnp.float32)
        vQ1 = srcg_ref[:, 1280 + C * m1:1280 + C * (m1 + 1)].astype(jnp.float32)
        vQ2 = srcg_ref[:, 1280 + C * m2:1280 + C * (m2 + 1)].astype(jnp.float32)
        om = (sP * sh1[:, m:m + 1] + vPm
              + vQ1 * sh1[:, m2:m2 + 1] - vQ2 * sh1[:, m1:m1 + 1])
        outs.append(om)
    for k in range(4):
        msg_ref[2 * k] = outs[k][:, :128]
        msg_ref[2 * k + 1] = outs[k][:, 128:]


def _edge_stage(srcg, dpg, prg, W1r, b1r, cen, wid):
    grid = (E // TE,)
    return pl.pallas_call(
        _k4_body,
        grid=grid,
        in_specs=[
            pl.BlockSpec((TE, SRC_W), lambda i: (i, 0)),
            pl.BlockSpec((TE, DST_W), lambda i: (i, 0)),
            pl.BlockSpec((TE, POS_W), lambda i: (i, 0)),
            pl.BlockSpec((NUM_RBF, C), lambda i: (0, 0)),
            pl.BlockSpec((1, C), lambda i: (0, 0)),
            pl.BlockSpec((1, NUM_RBF), lambda i: (0, 0)),
            pl.BlockSpec((1, NUM_RBF), lambda i: (0, 0)),
        ],
        out_specs=pl.BlockSpec((NGRP, TE, 128), lambda i: (0, i, 0)),
        out_shape=jax.ShapeDtypeStruct((NGRP, E, 128), jnp.float32),
    )(srcg, dpg, prg, W1r, b1r, cen, wid)


def _k6_body(m_ref, xs_ref, xvT_ref, w2_ref, b2_ref, outs_ref, outv_ref):
    ms = [m_ref[0, g] + m_ref[1, g] for g in range(NGRP)]
    H = jnp.concatenate(ms[0:2], axis=1)
    so = jnp.dot(H, w2_ref[...], preferred_element_type=jnp.float32) + b2_ref[...]
    so = so * (1.0 / (1.0 + jnp.exp(-so)))
    gates = 1.0 / (1.0 + jnp.exp(-so))
    outs_ref[...] = xs_ref[...] + so
    for m in range(3):
        vm = jnp.concatenate(ms[2 + 2 * m:4 + 2 * m], axis=1)
        outv_ref[m] = xvT_ref[m] + vm * gates


def _finish(parts, xs, xvT, W2, b2r):
    grid = (N // TN,)
    return pl.pallas_call(
        _k6_body,
        grid=grid,
        in_specs=[
            pl.BlockSpec((NC, NGRP, TN, 128), lambda i: (0, 0, i, 0)),
            pl.BlockSpec((TN, C), lambda i: (i, 0)),
            pl.BlockSpec((3, TN, C), lambda i: (0, i, 0)),
            pl.BlockSpec((C, C), lambda i: (0, 0)),
            pl.BlockSpec((1, C), lambda i: (0, 0)),
        ],
        out_specs=[
            pl.BlockSpec((TN, C), lambda i: (i, 0)),
            pl.BlockSpec((3, TN, C), lambda i: (0, i, 0)),
        ],
        out_shape=[
            jax.ShapeDtypeStruct((N, C), jnp.float32),
            jax.ShapeDtypeStruct((3, N, C), jnp.float32),
        ],
    )(parts, xs, xvT, W2, b2r)


@jax.jit
def kernel(x_scalar, x_vector, edge_index, edge_attr, pos, W1, b1, W2, b2,
           w00_0, w11_0, w01_1, w10_1, w11_1, centers, widths):
    del edge_attr, w00_0, w11_0
    xs = x_scalar
    xvT = jnp.transpose(x_vector, (2, 0, 1))
    pos16 = jnp.pad(pos, ((0, 0), (0, POS_W - 3)))
    row = edge_index[0]
    col = edge_index[1]
    row2 = row.reshape(NW, EPW)
    col2 = col.reshape(NW, EPW)
    col3 = col.reshape(NW, NWIN, GW)

    Wsc = jnp.concatenate([W1[:C], ALPHA1 * w01_1, W1[C:2 * C]], axis=1)
    Wv = jnp.concatenate([(ALPHA1 * SH0C) * w10_1,
                          (ALPHA1 / math.sqrt(2.0)) * w11_1], axis=1)
    W1r = W1[2 * C:]
    b1r = b1.reshape(1, C)
    b2r = b2.reshape(1, C)
    cen = centers.reshape(1, NUM_RBF)
    wid = widths.reshape(1, NUM_RBF)

    src_tab, dst_tab, pos_tab = _node_tables(xs, xvT, pos16, Wsc, Wv)

    srcg, dpg, prg = _k3_call(src_tab, dst_tab, pos_tab, row2, col2)

    msgs = _edge_stage(srcg, dpg, prg, W1r, b1r, cen, wid)

    zeros = jnp.zeros((NCHUNK, 128), jnp.float32)
    parts = _k5_call(msgs, col3, zeros)

    out_s, outvT = _finish(parts, xs, xvT, W2, b2r)
    out_v = jnp.transpose(outvT, (1, 2, 0))
    return (out_s, out_v)
